# baseline (speedup 1.0000x reference)
import functools

import jax
import jax.numpy as jnp
from jax import lax
from jax.experimental import pallas as pl
from jax.experimental.pallas import tpu as pltpu
from jax.experimental.pallas import tpu_sc as plsc

_N = 50000
_P = 375000
_H = 16

_BN = 2000
_BP = 3000
_NSTEP = _N // _BN
_PSTEP = _P // _BP
_NR = _BN // 8
_PR = _BP // 8

_GCH = 1500
_SCH = 750
_NW = 32


def _enc(x, s_sel, w1t, b1t, w2bd, b2t):
    xb = jnp.dot(x, s_sel, preferred_element_type=jnp.float32)
    h = jnp.maximum(xb * w1t + b1t, 0.0)
    return jnp.dot(h, w2bd, preferred_element_type=jnp.float32) + b2t


def _enc_pair_body(x1_ref, x2_ref, s_ref, w1_ref, b1_ref, w2_ref, b2_ref, o_ref):
    o_ref[0] = jnp.concatenate(
        [_enc(x1_ref[0], s_ref[...], w1_ref[...], b1_ref[...], w2_ref[...], b2_ref[...]),
         _enc(x2_ref[0], s_ref[...], w1_ref[...], b1_ref[...], w2_ref[...], b2_ref[...])],
        axis=1)


def _enc_diag_body(x_ref, s_ref, w1_ref, b1_ref, w2_ref, b2_ref, o_ref):
    o_ref[0] = _enc(x_ref[0], s_ref[...], w1_ref[...], b1_ref[...], w2_ref[...],
                    b2_ref[...])


def _enc_node_body(x_ref, s_ref, w1_ref, b1_ref, w2_ref, b2_ref, mb_ref, mc_ref,
                   o_ref, tbl_ref):
    nh = _enc(x_ref[0], s_ref[...], w1_ref[...], b1_ref[...], w2_ref[...],
              b2_ref[...])
    o_ref[0] = nh
    tbl_ref[0] = jnp.concatenate(
        [jnp.dot(nh, mb_ref[...], preferred_element_type=jnp.float32),
         jnp.dot(nh, mc_ref[...], preferred_element_type=jnp.float32)], axis=1)


def _pair_body(e1_ref, e2_ref, gbs_ref, gcd_ref, gbd_ref, gcs_ref,
               a_ref, b1_ref, w2_ref, b2_ref, o_ref):
    a = a_ref[...]
    w2 = w2_ref[...]
    b1 = b1_ref[...]
    b2 = b2_ref[...]
    m1 = gbs_ref[0] + gcd_ref[0]
    m2 = gbd_ref[0] + gcs_ref[0]
    h1 = jnp.maximum(jnp.dot(e1_ref[0], a, preferred_element_type=jnp.float32)
                     + m1 + b1, 0.0)
    h2 = jnp.maximum(jnp.dot(e2_ref[0], a, preferred_element_type=jnp.float32)
                     + m2 + b1, 0.0)
    o_ref[0] = jnp.concatenate(
        [jnp.dot(h1, w2, preferred_element_type=jnp.float32) + b2,
         jnp.dot(h2, w2, preferred_element_type=jnp.float32) + b2], axis=1)


def _diag_body(e0_ref, tbl_ref, a_ref, b1_ref, w2_ref, b2_ref, o_ref):
    tbl = tbl_ref[0]
    m0 = tbl[:, :128] + tbl[:, 128:]
    h = jnp.maximum(jnp.dot(e0_ref[0], a_ref[...], preferred_element_type=jnp.float32)
                    + m0 + b1_ref[...], 0.0)
    o_ref[0] = jnp.dot(h, w2_ref[...], preferred_element_type=jnp.float32) + b2_ref[...]


def _node_body(nh_ref, e0_ref, p0_ref, p1_ref, a_ref, g_ref, b1_ref, w2_ref,
               b2_ref, mb_ref, mc_ref, o_ref, tbl_ref):
    agg = e0_ref[0] + p0_ref[0, 0] + p1_ref[0, 0]
    h = jnp.maximum(jnp.dot(nh_ref[0], a_ref[...], preferred_element_type=jnp.float32)
                    + jnp.dot(agg, g_ref[...], preferred_element_type=jnp.float32)
                    + b1_ref[...], 0.0)
    nh = jnp.dot(h, w2_ref[...], preferred_element_type=jnp.float32) + b2_ref[...]
    o_ref[0] = nh
    tbl_ref[0] = jnp.concatenate(
        [jnp.dot(nh, mb_ref[...], preferred_element_type=jnp.float32),
         jnp.dot(nh, mc_ref[...], preferred_element_type=jnp.float32)], axis=1)


def _dec_body(e1_ref, e2_ref, w1_ref, b1_ref, w2_ref, b2_ref, o_ref):
    avg = 0.5 * (e1_ref[0] + e2_ref[0])
    h = jnp.maximum(jnp.dot(avg, w1_ref[...], preferred_element_type=jnp.float32)
                    + b1_ref[...], 0.0)
    o_ref[0] = jnp.dot(h, w2_ref[...], preferred_element_type=jnp.float32) + b2_ref[...]


_MESH = plsc.VectorSubcoreMesh(core_axis_name="c", subcore_axis_name="s")


@functools.partial(
    pl.kernel,
    mesh=_MESH,
    compiler_params=pltpu.CompilerParams(use_tc_tiling_on_sc=False),
    out_type=jax.ShapeDtypeStruct((_GCH, 1000, _H), jnp.float32),
    scratch_types=[
        pltpu.VMEM((1000,), jnp.int32),
        pltpu.VMEM((1000, _H), jnp.float32),
        pltpu.SemaphoreType.DMA,
    ],
)
def _sc_gather(tbl_hbm, idx_hbm, out_hbm, idx_v, rows_v, sem):
    wid = lax.axis_index("s") * 2 + lax.axis_index("c")

    def body(i, carry):
        c = wid + _NW * i

        @pl.when(c < _GCH)
        def _():
            pltpu.sync_copy(idx_hbm.at[c], idx_v)
            pltpu.async_copy(tbl_hbm.at[plsc.Indices(idx_v)], rows_v, sem).wait()
            pltpu.sync_copy(rows_v, out_hbm.at[c])

        return carry

    lax.fori_loop(0, (_GCH + _NW - 1) // _NW, body, 0)


@functools.partial(
    pl.kernel,
    mesh=_MESH,
    compiler_params=pltpu.CompilerParams(use_tc_tiling_on_sc=False),
    out_type=jax.ShapeDtypeStruct((2, 16, _N // 16, _H), jnp.float32),
    scratch_types=[
        pltpu.VMEM((1000,), jnp.int32),
        pltpu.VMEM((1000, _H), jnp.float32),
        pltpu.VMEM((_N // 16, _H), jnp.float32),
        pltpu.VMEM_SHARED((_N, _H), jnp.float32),
        pltpu.SemaphoreType.DMA,
    ],
)
def _sc_scatter(vals_hbm, idx_hbm, zeros_hbm, out_hbm, idx_v, vals_v, buf_v,
                acc_sh, sem):
    cid = lax.axis_index("c")
    sid = lax.axis_index("s")
    wid = sid * 2 + cid
    stripe = _N // 16

    pltpu.sync_copy(zeros_hbm.at[sid], acc_sh.at[pl.ds(sid * stripe, stripe)])
    plsc.subcore_barrier()

    def body(i, carry):
        c = wid + _NW * i

        @pl.when(c < _SCH)
        def _():
            pltpu.sync_copy(vals_hbm.at[c], vals_v)
            pltpu.sync_copy(idx_hbm.at[c], idx_v)
            pltpu.sync_copy(vals_v, acc_sh.at[plsc.Indices(idx_v)], add=True)

        return carry

    lax.fori_loop(0, (_SCH + _NW - 1) // _NW, body, 0)
    plsc.subcore_barrier()
    pltpu.sync_copy(acc_sh.at[pl.ds(sid * stripe, stripe)], buf_v)
    pltpu.sync_copy(buf_v, out_hbm.at[cid, sid])


def _full(shape):
    return pl.BlockSpec(shape, lambda i: tuple(0 for _ in shape))


def _rowmap(n):
    s = n // _BN
    rem = n % _BN
    return 16 * _NR * s + 16 * (rem // 8) + (rem % 8)


def kernel(nodes, edges, receivers, senders, bi_edges_indx,
           ne_W1, ne_b1, ne_W2, ne_b2,
           ee_W1, ee_b1, ee_W2, ee_b2,
           me_W1, me_b1, me_W2, me_b2,
           mn_W1, mn_b1, mn_W2, mn_b2,
           dec_W1, dec_b1, dec_W2, dec_b2):
    f32 = jnp.float32
    i32 = jnp.int32
    src = senders[_N:_N + _P]
    dst = receivers[_N:_N + _P]

    fsrc = _rowmap(src)
    fdst = _rowmap(dst)
    gidx = jnp.concatenate([fsrc, fdst + 8, fdst, fsrc + 8]).astype(i32)
    gidx = gidx.reshape(_GCH, 1000)
    t = jnp.arange(2 * _P)
    rem = t % (2 * _BP)
    u = rem % 16
    p = _BP * (t // (2 * _BP)) + 8 * (rem // 16) + (u % 8)
    sidx = jnp.where(u < 8, dst[p], src[p]).astype(i32).reshape(_SCH, 1000)
    zeros16 = jnp.zeros((16, _N // 16, _H), f32)

    eye8 = jnp.eye(8, dtype=f32)
    bd = lambda w: jnp.kron(eye8, w.astype(f32))
    tile8 = lambda b: jnp.tile(b.astype(f32), 8).reshape(1, -1)
    s_sel = jnp.kron(eye8, jnp.ones((1, _H), f32))
    meA = bd(me_W1[:_H])
    meB = bd(me_W1[_H:2 * _H])
    meC = bd(me_W1[2 * _H:])
    mnA = bd(mn_W1[:_H])
    mnG = bd(mn_W1[_H:])
    ne_w1t = tile8(ne_W1[0]); ne_b1t = tile8(ne_b1)
    ne_w2 = bd(ne_W2); ne_b2t = tile8(ne_b2)
    ee_w1t = tile8(ee_W1[0]); ee_b1t = tile8(ee_b1)
    ee_w2 = bd(ee_W2); ee_b2t = tile8(ee_b2)
    me_b1t = tile8(me_b1); me_w2 = bd(me_W2); me_b2t = tile8(me_b2)
    mn_b1t = tile8(mn_b1); mn_w2 = bd(mn_W2); mn_b2t = tile8(mn_b2)
    dc_w1 = bd(dec_W1); dc_b1t = tile8(dec_b1)
    dc_w2 = bd(dec_W2)
    dc_b2t = tile8(dec_b2)

    x1 = edges[_N:_N + _P].reshape(_PSTEP, _PR, 8)
    x2 = edges[_N + _P:].reshape(_PSTEP, _PR, 8)
    x0 = edges[:_N].reshape(_NSTEP, _NR, 8)
    xn = nodes.reshape(_NSTEP, _NR, 8)

    wenc = [_full((8, 128)), _full((1, 128)), _full((1, 128)),
            _full((128, 128)), _full((1, 128))]
    w16 = _full((128, 128))
    b16 = _full((1, 128))

    vee = pl.pallas_call(
        _enc_pair_body,
        grid=(_PSTEP,),
        in_specs=[pl.BlockSpec((1, _PR, 8), lambda i: (i, 0, 0)),
                  pl.BlockSpec((1, _PR, 8), lambda i: (i, 0, 0))] + wenc,
        out_specs=pl.BlockSpec((1, _PR, 256), lambda i: (i, 0, 0)),
        out_shape=jax.ShapeDtypeStruct((_PSTEP, _PR, 256), f32),
    )(x1, x2, s_sel, ee_w1t, ee_b1t, ee_w2, ee_b2t)

    e0 = pl.pallas_call(
        _enc_diag_body,
        grid=(_NSTEP,),
        in_specs=[pl.BlockSpec((1, _NR, 8), lambda i: (i, 0, 0))] + wenc,
        out_specs=pl.BlockSpec((1, _NR, 128), lambda i: (i, 0, 0)),
        out_shape=jax.ShapeDtypeStruct((_NSTEP, _NR, 128), f32),
    )(x0, s_sel, ee_w1t, ee_b1t, ee_w2, ee_b2t)

    nh, tbl = pl.pallas_call(
        _enc_node_body,
        grid=(_NSTEP,),
        in_specs=[pl.BlockSpec((1, _NR, 8), lambda i: (i, 0, 0))] + wenc
        + [w16, w16],
        out_specs=[pl.BlockSpec((1, _NR, 128), lambda i: (i, 0, 0)),
                   pl.BlockSpec((1, _NR, 256), lambda i: (i, 0, 0))],
        out_shape=[jax.ShapeDtypeStruct((_NSTEP, _NR, 128), f32),
                   jax.ShapeDtypeStruct((_NSTEP, _NR, 256), f32)],
    )(xn, s_sel, ne_w1t, ne_b1t, ne_w2, ne_b2t, meB, meC)

    pair_call = pl.pallas_call(
        _pair_body,
        grid=(_PSTEP,),
        in_specs=[pl.BlockSpec((1, _PR, 128), lambda i: (i, 0, 0)),
                  pl.BlockSpec((1, _PR, 128), lambda i: (i, 0, 1)),
                  pl.BlockSpec((1, _PR, 128), lambda i: (i, 0, 0)),
                  pl.BlockSpec((1, _PR, 128), lambda i: (i + _PSTEP, 0, 0)),
                  pl.BlockSpec((1, _PR, 128), lambda i: (i + 2 * _PSTEP, 0, 0)),
                  pl.BlockSpec((1, _PR, 128), lambda i: (i + 3 * _PSTEP, 0, 0)),
                  w16, b16, w16, b16],
        out_specs=pl.BlockSpec((1, _PR, 256), lambda i: (i, 0, 0)),
        out_shape=jax.ShapeDtypeStruct((_PSTEP, _PR, 256), f32),
    )
    diag_call = pl.pallas_call(
        _diag_body,
        grid=(_NSTEP,),
        in_specs=[pl.BlockSpec((1, _NR, 128), lambda i: (i, 0, 0)),
                  pl.BlockSpec((1, _NR, 256), lambda i: (i, 0, 0)),
                  w16, b16, w16, b16],
        out_specs=pl.BlockSpec((1, _NR, 128), lambda i: (i, 0, 0)),
        out_shape=jax.ShapeDtypeStruct((_NSTEP, _NR, 128), f32),
    )
    node_call = pl.pallas_call(
        _node_body,
        grid=(_NSTEP,),
        in_specs=[pl.BlockSpec((1, _NR, 128), lambda i: (i, 0, 0)),
                  pl.BlockSpec((1, _NR, 128), lambda i: (i, 0, 0)),
                  pl.BlockSpec((1, 1, _NR, 128), lambda i: (0, i, 0, 0)),
                  pl.BlockSpec((1, 1, _NR, 128), lambda i: (1, i, 0, 0)),
                  w16, w16, b16, w16, b16, w16, w16],
        out_specs=[pl.BlockSpec((1, _NR, 128), lambda i: (i, 0, 0)),
                   pl.BlockSpec((1, _NR, 256), lambda i: (i, 0, 0))],
        out_shape=[jax.ShapeDtypeStruct((_NSTEP, _NR, 128), f32),
                   jax.ShapeDtypeStruct((_NSTEP, _NR, 256), f32)],
    )

    for _ in range(3):
        g = tbl.reshape(2 * _N, _H)[gidx.reshape(-1)]
        g = g.reshape(4 * _PSTEP, _PR, 128)
        vee_n = pair_call(vee, vee, g, g, g, g, meA, me_b1t, me_w2, me_b2t)
        e0 = diag_call(e0, tbl, meA, me_b1t, me_w2, me_b2t)
        vee = vee_n
        ssum = jax.ops.segment_sum(vee.reshape(2 * _P, _H),
                                   sidx.reshape(-1), num_segments=_N)
        parts = jnp.stack([ssum, jnp.zeros_like(ssum)])
        parts = parts.reshape(2, _NSTEP, _NR, 128)
        nh, tbl = node_call(nh, e0, parts, parts, mnA, mnG, mn_b1t, mn_w2,
                            mn_b2t, meB, meC)

    dv = pl.pallas_call(
        _dec_body,
        grid=(_PSTEP,),
        in_specs=[pl.BlockSpec((1, _PR, 128), lambda i: (i, 0, 0)),
                  pl.BlockSpec((1, _PR, 128), lambda i: (i, 0, 1)),
                  w16, b16, _full((128, 8)), _full((1, 8))],
        out_specs=pl.BlockSpec((1, _PR, 8), lambda i: (i, 0, 0)),
        out_shape=jax.ShapeDtypeStruct((_PSTEP, _PR, 8), f32),
    )(vee, vee, dc_w1, dc_b1t, dc_w2, dc_b2t).reshape(_P)

    zi = jnp.zeros((), i32)
    t1 = dst >= src
    t2 = src >= dst
    data = jnp.concatenate([edges[:_N, 0],
                            jnp.where(t1, dv, 0.0),
                            jnp.where(t2, dv, 0.0)])
    rr = jnp.concatenate([receivers[:_N],
                          jnp.where(t1, dst, zi),
                          jnp.where(t2, src, zi)])
    ss = jnp.concatenate([senders[:_N],
                          jnp.where(t1, src, zi),
                          jnp.where(t2, dst, zi)])
    return data, jnp.stack([rr, ss], axis=1)


# speedup vs baseline: 18.0243x; 18.0243x over previous
import functools

import jax
import jax.numpy as jnp
from jax import lax
from jax.experimental import pallas as pl
from jax.experimental.pallas import tpu as pltpu
from jax.experimental.pallas import tpu_sc as plsc

_N = 50000
_P = 375000
_H = 16

_BN = 2000
_BP = 3000
_NSTEP = _N // _BN
_PSTEP = _P // _BP

_CH = 1000
_GCH = 2 * _P // _CH
_SCH = 2 * _P // _CH
_NW = 32


def _enc_t(x_row, w1t_ref, b1_ref, w2t_ref, b2_ref):
    ht = jnp.maximum(
        jnp.dot(w1t_ref[...], x_row, preferred_element_type=jnp.float32)
        + b1_ref[...], 0.0)
    return (jnp.dot(w2t_ref[...], ht, preferred_element_type=jnp.float32)
            + b2_ref[...]).T


def _enc_body(x_ref, w1t_ref, b1_ref, w2t_ref, b2_ref, o_ref):
    o_ref[...] = _enc_t(x_ref[0], w1t_ref, b1_ref, w2t_ref, b2_ref)


def _enc_node_body(x_ref, w1t_ref, b1_ref, w2t_ref, b2_ref, mb_ref, mc_ref,
                   o_ref, tbl_ref):
    nh = _enc_t(x_ref[0], w1t_ref, b1_ref, w2t_ref, b2_ref)
    o_ref[...] = nh
    tbl_ref[...] = jnp.concatenate(
        [jnp.dot(nh, mb_ref[...], preferred_element_type=jnp.float32),
         jnp.dot(nh, mc_ref[...], preferred_element_type=jnp.float32)], axis=1)


def _pair_body(e1_ref, e2_ref, gs_ref, gd_ref, a_ref, b1_ref, w2_ref, b2_ref,
               o1_ref, o2_ref):
    gs = gs_ref[...]
    gd = gd_ref[...]
    a = a_ref[...]
    w2 = w2_ref[...]
    b1 = b1_ref[...]
    b2 = b2_ref[...]
    m1 = gs[:, :_H] + gd[:, _H:]
    m2 = gd[:, :_H] + gs[:, _H:]
    h1 = jnp.maximum(jnp.dot(e1_ref[...], a, preferred_element_type=jnp.float32)
                     + m1 + b1, 0.0)
    h2 = jnp.maximum(jnp.dot(e2_ref[...], a, preferred_element_type=jnp.float32)
                     + m2 + b1, 0.0)
    o1_ref[...] = jnp.dot(h1, w2, preferred_element_type=jnp.float32) + b2
    o2_ref[...] = jnp.dot(h2, w2, preferred_element_type=jnp.float32) + b2


def _diag_body(e0_ref, tbl_ref, a_ref, b1_ref, w2_ref, b2_ref, o_ref):
    tbl = tbl_ref[...]
    m0 = tbl[:, :_H] + tbl[:, _H:]
    h = jnp.maximum(jnp.dot(e0_ref[...], a_ref[...],
                            preferred_element_type=jnp.float32)
                    + m0 + b1_ref[...], 0.0)
    o_ref[...] = jnp.dot(h, w2_ref[...], preferred_element_type=jnp.float32) \
        + b2_ref[...]


def _node_body(nh_ref, e0_ref, p0_ref, p1_ref, a_ref, g_ref, b1_ref, w2_ref,
               b2_ref, mb_ref, mc_ref, o_ref, tbl_ref):
    agg = e0_ref[...] + p0_ref[...] + p1_ref[...]
    h = jnp.maximum(jnp.dot(nh_ref[...], a_ref[...],
                            preferred_element_type=jnp.float32)
                    + jnp.dot(agg, g_ref[...], preferred_element_type=jnp.float32)
                    + b1_ref[...], 0.0)
    nh = jnp.dot(h, w2_ref[...], preferred_element_type=jnp.float32) + b2_ref[...]
    o_ref[...] = nh
    tbl_ref[...] = jnp.concatenate(
        [jnp.dot(nh, mb_ref[...], preferred_element_type=jnp.float32),
         jnp.dot(nh, mc_ref[...], preferred_element_type=jnp.float32)], axis=1)


def _dec_body(e1_ref, e2_ref, w1_ref, b1_ref, w2b_ref, b2_ref, o_ref):
    avg = 0.5 * (e1_ref[...] + e2_ref[...])
    h = jnp.maximum(jnp.dot(avg, w1_ref[...], preferred_element_type=jnp.float32)
                    + b1_ref[...], 0.0)
    o_ref[...] = jnp.dot(h, w2b_ref[...], preferred_element_type=jnp.float32) \
        + b2_ref[...]


_MESH = plsc.VectorSubcoreMesh(core_axis_name="c", subcore_axis_name="s")


@functools.partial(
    pl.kernel,
    mesh=_MESH,
    compiler_params=pltpu.CompilerParams(use_tc_tiling_on_sc=False),
    out_type=jax.ShapeDtypeStruct((2 * _P, 2 * _H), jnp.float32),
    scratch_types=[
        pltpu.VMEM((_CH,), jnp.int32),
        pltpu.VMEM((_CH, 2 * _H), jnp.float32),
        pltpu.SemaphoreType.DMA,
    ],
)
def _sc_gather(tbl_hbm, idx_hbm, out_hbm, idx_v, rows_v, sem):
    wid = lax.axis_index("s") * 2 + lax.axis_index("c")

    def body(i, carry):
        c = wid + _NW * i

        @pl.when(c < _GCH)
        def _():
            pltpu.sync_copy(idx_hbm.at[pl.ds(c * _CH, _CH)], idx_v)
            pltpu.async_copy(tbl_hbm.at[plsc.Indices(idx_v)], rows_v, sem).wait()
            pltpu.sync_copy(rows_v, out_hbm.at[pl.ds(c * _CH, _CH)])

        return carry

    lax.fori_loop(0, (_GCH + _NW - 1) // _NW, body, 0)


@functools.partial(
    pl.kernel,
    mesh=_MESH,
    compiler_params=pltpu.CompilerParams(use_tc_tiling_on_sc=False),
    out_type=jax.ShapeDtypeStruct((2 * _N, _H), jnp.float32),
    scratch_types=[
        pltpu.VMEM((_CH,), jnp.int32),
        pltpu.VMEM((_CH, _H), jnp.float32),
        pltpu.VMEM((_N // 16, _H), jnp.float32),
        pltpu.VMEM_SHARED((_N, _H), jnp.float32),
        pltpu.SemaphoreType.DMA,
    ],
)
def _sc_scatter(v1_hbm, i1_hbm, v2_hbm, i2_hbm, zeros_hbm, out_hbm,
                idx_v, vals_v, buf_v, acc_sh, sem):
    cid = lax.axis_index("c")
    sid = lax.axis_index("s")
    wid = sid * 2 + cid
    stripe = _N // 16
    half = _SCH // 2

    pltpu.sync_copy(zeros_hbm.at[pl.ds(sid * stripe, stripe)],
                    acc_sh.at[pl.ds(sid * stripe, stripe)])
    plsc.subcore_barrier()

    def body(i, carry):
        c = wid + _NW * i

        @pl.when(c < half)
        def _():
            pltpu.sync_copy(v1_hbm.at[pl.ds(c * _CH, _CH)], vals_v)
            pltpu.sync_copy(i1_hbm.at[pl.ds(c * _CH, _CH)], idx_v)
            pltpu.sync_copy(vals_v, acc_sh.at[plsc.Indices(idx_v)], add=True)

        @pl.when(jnp.logical_and(c >= half, c < _SCH))
        def _():
            cc = c - half
            pltpu.sync_copy(v2_hbm.at[pl.ds(cc * _CH, _CH)], vals_v)
            pltpu.sync_copy(i2_hbm.at[pl.ds(cc * _CH, _CH)], idx_v)
            pltpu.sync_copy(vals_v, acc_sh.at[plsc.Indices(idx_v)], add=True)

        return carry

    lax.fori_loop(0, (_SCH + _NW - 1) // _NW, body, 0)
    plsc.subcore_barrier()
    pltpu.sync_copy(acc_sh.at[pl.ds(sid * stripe, stripe)], buf_v)
    pltpu.sync_copy(buf_v, out_hbm.at[pl.ds(cid * _N + sid * stripe, stripe)])


def _full(shape):
    return pl.BlockSpec(shape, lambda i: tuple(0 for _ in shape))


def kernel(nodes, edges, receivers, senders, bi_edges_indx,
           ne_W1, ne_b1, ne_W2, ne_b2,
           ee_W1, ee_b1, ee_W2, ee_b2,
           me_W1, me_b1, me_W2, me_b2,
           mn_W1, mn_b1, mn_W2, mn_b2,
           dec_W1, dec_b1, dec_W2, dec_b2):
    f32 = jnp.float32
    i32 = jnp.int32
    src = senders[_N:_N + _P]
    dst = receivers[_N:_N + _P]
    sd = jnp.concatenate([src, dst]).astype(i32)
    zeros = jnp.zeros((_N, _H), f32)

    meA = me_W1[:_H]
    meB = me_W1[_H:2 * _H]
    meC = me_W1[2 * _H:]
    mnA = mn_W1[:_H]
    mnG = mn_W1[_H:]
    me_b1r = me_b1.reshape(1, _H)
    me_b2r = me_b2.reshape(1, _H)
    mn_b1r = mn_b1.reshape(1, _H)
    mn_b2r = mn_b2.reshape(1, _H)
    dc_b1r = dec_b1.reshape(1, _H)
    dc_w2b = jnp.tile(dec_W2, (1, _H))
    dc_b2r = jnp.tile(dec_b2.reshape(1, 1), (1, _H))
    ne_w1t = ne_W1.T
    ne_b1c = ne_b1.reshape(_H, 1)
    ne_w2t = ne_W2.T
    ne_b2c = ne_b2.reshape(_H, 1)
    ee_w1t = ee_W1.T
    ee_b1c = ee_b1.reshape(_H, 1)
    ee_w2t = ee_W2.T
    ee_b2c = ee_b2.reshape(_H, 1)

    x1 = edges[_N:_N + _P].reshape(_PSTEP, 1, _BP)
    x2 = edges[_N + _P:].reshape(_PSTEP, 1, _BP)
    x0 = edges[:_N].reshape(_NSTEP, 1, _BN)
    xn = nodes.reshape(_NSTEP, 1, _BN)

    w16 = _full((_H, _H))
    b16 = _full((1, _H))
    wenc = [_full((_H, 1)), _full((_H, 1)), _full((_H, _H)), _full((_H, 1))]

    def rowspec(block, width):
        return pl.BlockSpec((block, width), lambda i: (i, 0))

    enc_pair = pl.pallas_call(
        _enc_body,
        grid=(_PSTEP,),
        in_specs=[pl.BlockSpec((1, 1, _BP), lambda i: (i, 0, 0))] + wenc,
        out_specs=rowspec(_BP, _H),
        out_shape=jax.ShapeDtypeStruct((_P, _H), f32),
    )
    e1 = enc_pair(x1, ee_w1t, ee_b1c, ee_w2t, ee_b2c)
    e2 = enc_pair(x2, ee_w1t, ee_b1c, ee_w2t, ee_b2c)
    e0 = pl.pallas_call(
        _enc_body,
        grid=(_NSTEP,),
        in_specs=[pl.BlockSpec((1, 1, _BN), lambda i: (i, 0, 0))] + wenc,
        out_specs=rowspec(_BN, _H),
        out_shape=jax.ShapeDtypeStruct((_N, _H), f32),
    )(x0, ee_w1t, ee_b1c, ee_w2t, ee_b2c)

    nh, tbl = pl.pallas_call(
        _enc_node_body,
        grid=(_NSTEP,),
        in_specs=[pl.BlockSpec((1, 1, _BN), lambda i: (i, 0, 0))] + wenc
        + [w16, w16],
        out_specs=[rowspec(_BN, _H), rowspec(_BN, 2 * _H)],
        out_shape=[jax.ShapeDtypeStruct((_N, _H), f32),
                   jax.ShapeDtypeStruct((_N, 2 * _H), f32)],
    )(xn, ne_w1t, ne_b1c, ne_w2t, ne_b2c, meB, meC)

    pair_call = pl.pallas_call(
        _pair_body,
        grid=(_PSTEP,),
        in_specs=[rowspec(_BP, _H), rowspec(_BP, _H),
                  pl.BlockSpec((_BP, 2 * _H), lambda i: (i, 0)),
                  pl.BlockSpec((_BP, 2 * _H), lambda i: (i + _PSTEP, 0)),
                  w16, b16, w16, b16],
        out_specs=[rowspec(_BP, _H), rowspec(_BP, _H)],
        out_shape=[jax.ShapeDtypeStruct((_P, _H), f32),
                   jax.ShapeDtypeStruct((_P, _H), f32)],
    )
    diag_call = pl.pallas_call(
        _diag_body,
        grid=(_NSTEP,),
        in_specs=[rowspec(_BN, _H), rowspec(_BN, 2 * _H),
                  w16, b16, w16, b16],
        out_specs=rowspec(_BN, _H),
        out_shape=jax.ShapeDtypeStruct((_N, _H), f32),
    )
    node_call = pl.pallas_call(
        _node_body,
        grid=(_NSTEP,),
        in_specs=[rowspec(_BN, _H), rowspec(_BN, _H),
                  pl.BlockSpec((_BN, _H), lambda i: (i, 0)),
                  pl.BlockSpec((_BN, _H), lambda i: (i + _NSTEP, 0)),
                  w16, w16, b16, w16, b16, w16, w16],
        out_specs=[rowspec(_BN, _H), rowspec(_BN, 2 * _H)],
        out_shape=[jax.ShapeDtypeStruct((_N, _H), f32),
                   jax.ShapeDtypeStruct((_N, 2 * _H), f32)],
    )

    for _ in range(3):
        g = _sc_gather(tbl, sd)
        e1n, e2n = pair_call(e1, e2, g, g, meA, me_b1r, me_W2, me_b2r)
        e0 = diag_call(e0, tbl, meA, me_b1r, me_W2, me_b2r)
        e1, e2 = e1n, e2n
        parts = _sc_scatter(e1, dst, e2, src, zeros)
        nh, tbl = node_call(nh, e0, parts, parts, mnA, mnG, mn_b1r, mn_W2,
                            mn_b2r, meB, meC)

    dv = pl.pallas_call(
        _dec_body,
        grid=(_PSTEP,),
        in_specs=[rowspec(_BP, _H), rowspec(_BP, _H),
                  w16, b16, w16, b16],
        out_specs=rowspec(_BP, _H),
        out_shape=jax.ShapeDtypeStruct((_P, _H), f32),
    )(e1, e2, dec_W1, dc_b1r, dc_w2b, dc_b2r)[:, 0]

    zi = jnp.zeros((), i32)
    t1 = dst >= src
    t2 = src >= dst
    data = jnp.concatenate([edges[:_N, 0],
                            jnp.where(t1, dv, 0.0),
                            jnp.where(t2, dv, 0.0)])
    rr = jnp.concatenate([receivers[:_N],
                          jnp.where(t1, dst, zi),
                          jnp.where(t2, src, zi)])
    ss = jnp.concatenate([senders[:_N],
                          jnp.where(t1, src, zi),
                          jnp.where(t2, dst, zi)])
    return data, jnp.stack([rr, ss], axis=1)


# speedup vs baseline: 18.2608x; 1.0131x over previous
import functools

import jax
import jax.numpy as jnp
from jax import lax
from jax.experimental import pallas as pl
from jax.experimental.pallas import tpu as pltpu
from jax.experimental.pallas import tpu_sc as plsc

_N = 50000
_P = 375000
_H = 16

_BN = 2000
_BP = 5000
_NSTEP = _N // _BN
_PSTEP = _P // _BP

_CH = 1000
_GCH = 2 * _P // _CH
_SCH = 2 * _P // _CH
_NW = 32


def _enc_t(x_row, w1t_ref, b1_ref, w2t_ref, b2_ref):
    ht = jnp.maximum(
        jnp.dot(w1t_ref[...], x_row, preferred_element_type=jnp.float32)
        + b1_ref[...], 0.0)
    return (jnp.dot(w2t_ref[...], ht, preferred_element_type=jnp.float32)
            + b2_ref[...]).T


def _enc_body(x_ref, w1t_ref, b1_ref, w2t_ref, b2_ref, o_ref):
    o_ref[...] = _enc_t(x_ref[0], w1t_ref, b1_ref, w2t_ref, b2_ref)


def _enc_node_body(x_ref, w1t_ref, b1_ref, w2t_ref, b2_ref, mb_ref, mc_ref,
                   o_ref, tbl_ref):
    nh = _enc_t(x_ref[0], w1t_ref, b1_ref, w2t_ref, b2_ref)
    o_ref[...] = nh
    tbl_ref[...] = jnp.concatenate(
        [jnp.dot(nh, mb_ref[...], preferred_element_type=jnp.float32),
         jnp.dot(nh, mc_ref[...], preferred_element_type=jnp.float32)], axis=1)


def _pair_body(e1_ref, e2_ref, gs_ref, gd_ref, a_ref, b1_ref, w2_ref, b2_ref,
               o1_ref, o2_ref):
    gs = gs_ref[...]
    gd = gd_ref[...]
    a = a_ref[...]
    w2 = w2_ref[...]
    b1 = b1_ref[...]
    b2 = b2_ref[...]
    m1 = gs[:, :_H] + gd[:, _H:]
    m2 = gd[:, :_H] + gs[:, _H:]
    h1 = jnp.maximum(jnp.dot(e1_ref[...], a, preferred_element_type=jnp.float32)
                     + m1 + b1, 0.0)
    h2 = jnp.maximum(jnp.dot(e2_ref[...], a, preferred_element_type=jnp.float32)
                     + m2 + b1, 0.0)
    o1_ref[...] = jnp.dot(h1, w2, preferred_element_type=jnp.float32) + b2
    o2_ref[...] = jnp.dot(h2, w2, preferred_element_type=jnp.float32) + b2


def _diag_body(e0_ref, tbl_ref, a_ref, b1_ref, w2_ref, b2_ref, o_ref):
    tbl = tbl_ref[...]
    m0 = tbl[:, :_H] + tbl[:, _H:]
    h = jnp.maximum(jnp.dot(e0_ref[...], a_ref[...],
                            preferred_element_type=jnp.float32)
                    + m0 + b1_ref[...], 0.0)
    o_ref[...] = jnp.dot(h, w2_ref[...], preferred_element_type=jnp.float32) \
        + b2_ref[...]


def _node_body(nh_ref, e0_ref, tbl_ref, p0_ref, p1_ref,
               ea_ref, eb1_ref, ew2_ref, eb2_ref,
               a_ref, g_ref, b1_ref, w2_ref, b2_ref, mb_ref, mc_ref,
               o_ref, tbl_o_ref, e0_o_ref):
    tbl = tbl_ref[...]
    m0 = tbl[:, :_H] + tbl[:, _H:]
    h0 = jnp.maximum(jnp.dot(e0_ref[...], ea_ref[...],
                             preferred_element_type=jnp.float32)
                     + m0 + eb1_ref[...], 0.0)
    e0n = jnp.dot(h0, ew2_ref[...], preferred_element_type=jnp.float32) \
        + eb2_ref[...]
    e0_o_ref[...] = e0n
    agg = e0n + p0_ref[...] + p1_ref[...]
    h = jnp.maximum(jnp.dot(nh_ref[...], a_ref[...],
                            preferred_element_type=jnp.float32)
                    + jnp.dot(agg, g_ref[...], preferred_element_type=jnp.float32)
                    + b1_ref[...], 0.0)
    nh = jnp.dot(h, w2_ref[...], preferred_element_type=jnp.float32) + b2_ref[...]
    o_ref[...] = nh
    tbl_o_ref[...] = jnp.concatenate(
        [jnp.dot(nh, mb_ref[...], preferred_element_type=jnp.float32),
         jnp.dot(nh, mc_ref[...], preferred_element_type=jnp.float32)], axis=1)


def _dec_body(e1_ref, e2_ref, w1_ref, b1_ref, w2b_ref, b2_ref, o_ref):
    avg = 0.5 * (e1_ref[...] + e2_ref[...])
    h = jnp.maximum(jnp.dot(avg, w1_ref[...], preferred_element_type=jnp.float32)
                    + b1_ref[...], 0.0)
    o_ref[...] = jnp.dot(h, w2b_ref[...], preferred_element_type=jnp.float32) \
        + b2_ref[...]


_MESH = plsc.VectorSubcoreMesh(core_axis_name="c", subcore_axis_name="s")


@functools.partial(
    pl.kernel,
    mesh=_MESH,
    compiler_params=pltpu.CompilerParams(use_tc_tiling_on_sc=False),
    out_type=jax.ShapeDtypeStruct((2 * _P, 2 * _H), jnp.float32),
    scratch_types=[
        pltpu.VMEM((_CH,), jnp.int32),
        pltpu.VMEM((_CH, 2 * _H), jnp.float32),
        pltpu.SemaphoreType.DMA,
    ],
)
def _sc_gather(tbl_hbm, idx_hbm, out_hbm, idx_v, rows_v, sem):
    wid = lax.axis_index("s") * 2 + lax.axis_index("c")

    def body(i, carry):
        c = wid + _NW * i

        @pl.when(c < _GCH)
        def _():
            pltpu.sync_copy(idx_hbm.at[pl.ds(c * _CH, _CH)], idx_v)
            pltpu.async_copy(tbl_hbm.at[plsc.Indices(idx_v)], rows_v, sem).wait()
            pltpu.sync_copy(rows_v, out_hbm.at[pl.ds(c * _CH, _CH)])

        return carry

    lax.fori_loop(0, (_GCH + _NW - 1) // _NW, body, 0)


@functools.partial(
    pl.kernel,
    mesh=_MESH,
    compiler_params=pltpu.CompilerParams(use_tc_tiling_on_sc=False),
    out_type=jax.ShapeDtypeStruct((2 * _N, _H), jnp.float32),
    scratch_types=[
        pltpu.VMEM((_CH,), jnp.int32),
        pltpu.VMEM((_CH, _H), jnp.float32),
        pltpu.VMEM((_N // 16, _H), jnp.float32),
        pltpu.VMEM_SHARED((_N, _H), jnp.float32),
        pltpu.SemaphoreType.DMA,
    ],
)
def _sc_scatter(v1_hbm, i1_hbm, v2_hbm, i2_hbm, zeros_hbm, out_hbm,
                idx_v, vals_v, buf_v, acc_sh, sem):
    cid = lax.axis_index("c")
    sid = lax.axis_index("s")
    wid = sid * 2 + cid
    stripe = _N // 16
    half = _SCH // 2

    pltpu.sync_copy(zeros_hbm.at[pl.ds(sid * stripe, stripe)],
                    acc_sh.at[pl.ds(sid * stripe, stripe)])
    plsc.subcore_barrier()

    def body(i, carry):
        c = wid + _NW * i

        @pl.when(c < half)
        def _():
            pltpu.sync_copy(v1_hbm.at[pl.ds(c * _CH, _CH)], vals_v)
            pltpu.sync_copy(i1_hbm.at[pl.ds(c * _CH, _CH)], idx_v)
            pltpu.sync_copy(vals_v, acc_sh.at[plsc.Indices(idx_v)], add=True)

        @pl.when(jnp.logical_and(c >= half, c < _SCH))
        def _():
            cc = c - half
            pltpu.sync_copy(v2_hbm.at[pl.ds(cc * _CH, _CH)], vals_v)
            pltpu.sync_copy(i2_hbm.at[pl.ds(cc * _CH, _CH)], idx_v)
            pltpu.sync_copy(vals_v, acc_sh.at[plsc.Indices(idx_v)], add=True)

        return carry

    lax.fori_loop(0, (_SCH + _NW - 1) // _NW, body, 0)
    plsc.subcore_barrier()
    pltpu.sync_copy(acc_sh.at[pl.ds(sid * stripe, stripe)], buf_v)
    pltpu.sync_copy(buf_v, out_hbm.at[pl.ds(cid * _N + sid * stripe, stripe)])


def _full(shape):
    return pl.BlockSpec(shape, lambda i: tuple(0 for _ in shape))


def kernel(nodes, edges, receivers, senders, bi_edges_indx,
           ne_W1, ne_b1, ne_W2, ne_b2,
           ee_W1, ee_b1, ee_W2, ee_b2,
           me_W1, me_b1, me_W2, me_b2,
           mn_W1, mn_b1, mn_W2, mn_b2,
           dec_W1, dec_b1, dec_W2, dec_b2):
    f32 = jnp.float32
    i32 = jnp.int32
    src = senders[_N:_N + _P]
    dst = receivers[_N:_N + _P]
    sd = jnp.concatenate([src, dst]).astype(i32)
    zeros = jnp.zeros((_N, _H), f32)

    meA = me_W1[:_H]
    meB = me_W1[_H:2 * _H]
    meC = me_W1[2 * _H:]
    mnA = mn_W1[:_H]
    mnG = mn_W1[_H:]
    me_b1r = me_b1.reshape(1, _H)
    me_b2r = me_b2.reshape(1, _H)
    mn_b1r = mn_b1.reshape(1, _H)
    mn_b2r = mn_b2.reshape(1, _H)
    dc_b1r = dec_b1.reshape(1, _H)
    dc_w2b = jnp.tile(dec_W2, (1, _H))
    dc_b2r = jnp.tile(dec_b2.reshape(1, 1), (1, _H))
    ne_w1t = ne_W1.T
    ne_b1c = ne_b1.reshape(_H, 1)
    ne_w2t = ne_W2.T
    ne_b2c = ne_b2.reshape(_H, 1)
    ee_w1t = ee_W1.T
    ee_b1c = ee_b1.reshape(_H, 1)
    ee_w2t = ee_W2.T
    ee_b2c = ee_b2.reshape(_H, 1)

    x1 = edges[_N:_N + _P].reshape(_PSTEP, 1, _BP)
    x2 = edges[_N + _P:].reshape(_PSTEP, 1, _BP)
    x0 = edges[:_N].reshape(_NSTEP, 1, _BN)
    xn = nodes.reshape(_NSTEP, 1, _BN)

    w16 = _full((_H, _H))
    b16 = _full((1, _H))
    wenc = [_full((_H, 1)), _full((_H, 1)), _full((_H, _H)), _full((_H, 1))]

    def rowspec(block, width):
        return pl.BlockSpec((block, width), lambda i: (i, 0))

    enc_pair = pl.pallas_call(
        _enc_body,
        grid=(_PSTEP,),
        in_specs=[pl.BlockSpec((1, 1, _BP), lambda i: (i, 0, 0))] + wenc,
        out_specs=rowspec(_BP, _H),
        out_shape=jax.ShapeDtypeStruct((_P, _H), f32),
    )
    e1 = enc_pair(x1, ee_w1t, ee_b1c, ee_w2t, ee_b2c)
    e2 = enc_pair(x2, ee_w1t, ee_b1c, ee_w2t, ee_b2c)
    e0 = pl.pallas_call(
        _enc_body,
        grid=(_NSTEP,),
        in_specs=[pl.BlockSpec((1, 1, _BN), lambda i: (i, 0, 0))] + wenc,
        out_specs=rowspec(_BN, _H),
        out_shape=jax.ShapeDtypeStruct((_N, _H), f32),
    )(x0, ee_w1t, ee_b1c, ee_w2t, ee_b2c)

    nh, tbl = pl.pallas_call(
        _enc_node_body,
        grid=(_NSTEP,),
        in_specs=[pl.BlockSpec((1, 1, _BN), lambda i: (i, 0, 0))] + wenc
        + [w16, w16],
        out_specs=[rowspec(_BN, _H), rowspec(_BN, 2 * _H)],
        out_shape=[jax.ShapeDtypeStruct((_N, _H), f32),
                   jax.ShapeDtypeStruct((_N, 2 * _H), f32)],
    )(xn, ne_w1t, ne_b1c, ne_w2t, ne_b2c, meB, meC)

    pair_call = pl.pallas_call(
        _pair_body,
        grid=(_PSTEP,),
        in_specs=[rowspec(_BP, _H), rowspec(_BP, _H),
                  pl.BlockSpec((_BP, 2 * _H), lambda i: (i, 0)),
                  pl.BlockSpec((_BP, 2 * _H), lambda i: (i + _PSTEP, 0)),
                  w16, b16, w16, b16],
        out_specs=[rowspec(_BP, _H), rowspec(_BP, _H)],
        out_shape=[jax.ShapeDtypeStruct((_P, _H), f32),
                   jax.ShapeDtypeStruct((_P, _H), f32)],
    )
    node_call = pl.pallas_call(
        _node_body,
        grid=(_NSTEP,),
        in_specs=[rowspec(_BN, _H), rowspec(_BN, _H), rowspec(_BN, 2 * _H),
                  pl.BlockSpec((_BN, _H), lambda i: (i, 0)),
                  pl.BlockSpec((_BN, _H), lambda i: (i + _NSTEP, 0)),
                  w16, b16, w16, b16,
                  w16, w16, b16, w16, b16, w16, w16],
        out_specs=[rowspec(_BN, _H), rowspec(_BN, 2 * _H), rowspec(_BN, _H)],
        out_shape=[jax.ShapeDtypeStruct((_N, _H), f32),
                   jax.ShapeDtypeStruct((_N, 2 * _H), f32),
                   jax.ShapeDtypeStruct((_N, _H), f32)],
    )

    for _ in range(3):
        g = _sc_gather(tbl, sd)
        e1n, e2n = pair_call(e1, e2, g, g, meA, me_b1r, me_W2, me_b2r)
        parts = _sc_scatter(e1n, dst, e2n, src, zeros)
        nh, tbl, e0 = node_call(nh, e0, tbl, parts, parts,
                                meA, me_b1r, me_W2, me_b2r,
                                mnA, mnG, mn_b1r, mn_W2, mn_b2r, meB, meC)
        e1, e2 = e1n, e2n

    dv = pl.pallas_call(
        _dec_body,
        grid=(_PSTEP,),
        in_specs=[rowspec(_BP, _H), rowspec(_BP, _H),
                  w16, b16, w16, b16],
        out_specs=rowspec(_BP, _H),
        out_shape=jax.ShapeDtypeStruct((_P, _H), f32),
    )(e1, e2, dec_W1, dc_b1r, dc_w2b, dc_b2r)[:, 0]

    zi = jnp.zeros((), i32)
    t1 = dst >= src
    t2 = src >= dst
    data = jnp.concatenate([edges[:_N, 0],
                            jnp.where(t1, dv, 0.0),
                            jnp.where(t2, dv, 0.0)])
    rr = jnp.concatenate([receivers[:_N],
                          jnp.where(t1, dst, zi),
                          jnp.where(t2, src, zi)])
    ss = jnp.concatenate([senders[:_N],
                          jnp.where(t1, src, zi),
                          jnp.where(t2, dst, zi)])
    return data, jnp.stack([rr, ss], axis=1)


# speedup vs baseline: 18.4699x; 1.0115x over previous
import functools

import jax
import jax.numpy as jnp
from jax import lax
from jax.experimental import pallas as pl
from jax.experimental.pallas import tpu as pltpu
from jax.experimental.pallas import tpu_sc as plsc

_N = 50000
_P = 375000
_H = 16

_BN = 2000
_BP = 5000
_NSTEP = _N // _BN
_PSTEP = _P // _BP

_GC = 3000
_CH = 1000
_GCH = 2 * _P // _GC
_SCH = 2 * _P // _CH
_NW = 32


def _enc_t(x_row, w1t_ref, b1_ref, w2t_ref, b2_ref):
    ht = jnp.maximum(
        jnp.dot(w1t_ref[...], x_row, preferred_element_type=jnp.float32)
        + b1_ref[...], 0.0)
    return (jnp.dot(w2t_ref[...], ht, preferred_element_type=jnp.float32)
            + b2_ref[...]).T


def _enc_body(x_ref, w1t_ref, b1_ref, w2t_ref, b2_ref, o_ref):
    o_ref[...] = _enc_t(x_ref[0], w1t_ref, b1_ref, w2t_ref, b2_ref)


def _enc_node_body(x_ref, w1t_ref, b1_ref, w2t_ref, b2_ref, mb_ref, mc_ref,
                   o_ref, tbl_ref):
    nh = _enc_t(x_ref[0], w1t_ref, b1_ref, w2t_ref, b2_ref)
    o_ref[...] = nh
    tbl_ref[...] = jnp.concatenate(
        [jnp.dot(nh, mb_ref[...], preferred_element_type=jnp.float32),
         jnp.dot(nh, mc_ref[...], preferred_element_type=jnp.float32)], axis=1)


def _pair_body(e1_ref, e2_ref, gs_ref, gd_ref, a_ref, b1_ref, w2_ref, b2_ref,
               o1_ref, o2_ref):
    gs = gs_ref[...]
    gd = gd_ref[...]
    a = a_ref[...]
    w2 = w2_ref[...]
    b1 = b1_ref[...]
    b2 = b2_ref[...]
    m1 = gs[:, :_H] + gd[:, _H:]
    m2 = gd[:, :_H] + gs[:, _H:]
    h1 = jnp.maximum(jnp.dot(e1_ref[...], a, preferred_element_type=jnp.float32)
                     + m1 + b1, 0.0)
    h2 = jnp.maximum(jnp.dot(e2_ref[...], a, preferred_element_type=jnp.float32)
                     + m2 + b1, 0.0)
    o1_ref[...] = jnp.dot(h1, w2, preferred_element_type=jnp.float32) + b2
    o2_ref[...] = jnp.dot(h2, w2, preferred_element_type=jnp.float32) + b2


def _diag_body(e0_ref, tbl_ref, a_ref, b1_ref, w2_ref, b2_ref, o_ref):
    tbl = tbl_ref[...]
    m0 = tbl[:, :_H] + tbl[:, _H:]
    h = jnp.maximum(jnp.dot(e0_ref[...], a_ref[...],
                            preferred_element_type=jnp.float32)
                    + m0 + b1_ref[...], 0.0)
    o_ref[...] = jnp.dot(h, w2_ref[...], preferred_element_type=jnp.float32) \
        + b2_ref[...]


def _node_body(nh_ref, e0_ref, tbl_ref, p0_ref, p1_ref,
               ea_ref, eb1_ref, ew2_ref, eb2_ref,
               a_ref, g_ref, b1_ref, w2_ref, b2_ref, mb_ref, mc_ref,
               o_ref, tbl_o_ref, e0_o_ref):
    tbl = tbl_ref[...]
    m0 = tbl[:, :_H] + tbl[:, _H:]
    h0 = jnp.maximum(jnp.dot(e0_ref[...], ea_ref[...],
                             preferred_element_type=jnp.float32)
                     + m0 + eb1_ref[...], 0.0)
    e0n = jnp.dot(h0, ew2_ref[...], preferred_element_type=jnp.float32) \
        + eb2_ref[...]
    e0_o_ref[...] = e0n
    agg = e0n + p0_ref[...] + p1_ref[...]
    h = jnp.maximum(jnp.dot(nh_ref[...], a_ref[...],
                            preferred_element_type=jnp.float32)
                    + jnp.dot(agg, g_ref[...], preferred_element_type=jnp.float32)
                    + b1_ref[...], 0.0)
    nh = jnp.dot(h, w2_ref[...], preferred_element_type=jnp.float32) + b2_ref[...]
    o_ref[...] = nh
    tbl_o_ref[...] = jnp.concatenate(
        [jnp.dot(nh, mb_ref[...], preferred_element_type=jnp.float32),
         jnp.dot(nh, mc_ref[...], preferred_element_type=jnp.float32)], axis=1)


def _dec_body(e1_ref, e2_ref, w1_ref, b1_ref, w2b_ref, b2_ref, o_ref):
    avg = 0.5 * (e1_ref[...] + e2_ref[...])
    h = jnp.maximum(jnp.dot(avg, w1_ref[...], preferred_element_type=jnp.float32)
                    + b1_ref[...], 0.0)
    o_ref[...] = jnp.dot(h, w2b_ref[...], preferred_element_type=jnp.float32) \
        + b2_ref[...]


_MESH = plsc.VectorSubcoreMesh(core_axis_name="c", subcore_axis_name="s")


@functools.partial(
    pl.kernel,
    mesh=_MESH,
    compiler_params=pltpu.CompilerParams(use_tc_tiling_on_sc=False),
    out_type=jax.ShapeDtypeStruct((2 * _P, 2 * _H), jnp.float32),
    scratch_types=[
        pltpu.VMEM((_GC,), jnp.int32),
        pltpu.VMEM((_GC, 2 * _H), jnp.float32),
        pltpu.SemaphoreType.DMA,
    ],
)
def _sc_gather(tbl_hbm, idx_hbm, out_hbm, idx_v, rows_v, sem):
    wid = lax.axis_index("s") * 2 + lax.axis_index("c")

    def body(i, carry):
        c = wid + _NW * i

        @pl.when(c < _GCH)
        def _():
            pltpu.sync_copy(idx_hbm.at[pl.ds(c * _GC, _GC)], idx_v)
            pltpu.async_copy(tbl_hbm.at[plsc.Indices(idx_v)], rows_v, sem).wait()
            pltpu.sync_copy(rows_v, out_hbm.at[pl.ds(c * _GC, _GC)])

        return carry

    lax.fori_loop(0, (_GCH + _NW - 1) // _NW, body, 0)


@functools.partial(
    pl.kernel,
    mesh=_MESH,
    compiler_params=pltpu.CompilerParams(use_tc_tiling_on_sc=False),
    out_type=jax.ShapeDtypeStruct((2 * _N, _H), jnp.float32),
    scratch_types=[
        pltpu.VMEM((_CH,), jnp.int32),
        pltpu.VMEM((_CH, _H), jnp.float32),
        pltpu.VMEM((_N // 16, _H), jnp.float32),
        pltpu.VMEM_SHARED((_N, _H), jnp.float32),
        pltpu.SemaphoreType.DMA,
    ],
)
def _sc_scatter(v1_hbm, i1_hbm, v2_hbm, i2_hbm, zeros_hbm, out_hbm,
                idx_v, vals_v, buf_v, acc_sh, sem):
    cid = lax.axis_index("c")
    sid = lax.axis_index("s")
    wid = sid * 2 + cid
    stripe = _N // 16
    half = _SCH // 2

    pltpu.sync_copy(zeros_hbm.at[pl.ds(sid * stripe, stripe)],
                    acc_sh.at[pl.ds(sid * stripe, stripe)])
    plsc.subcore_barrier()

    def body(i, carry):
        c = wid + _NW * i

        @pl.when(c < half)
        def _():
            pltpu.sync_copy(v1_hbm.at[pl.ds(c * _CH, _CH)], vals_v)
            pltpu.sync_copy(i1_hbm.at[pl.ds(c * _CH, _CH)], idx_v)
            pltpu.sync_copy(vals_v, acc_sh.at[plsc.Indices(idx_v)], add=True)

        @pl.when(jnp.logical_and(c >= half, c < _SCH))
        def _():
            cc = c - half
            pltpu.sync_copy(v2_hbm.at[pl.ds(cc * _CH, _CH)], vals_v)
            pltpu.sync_copy(i2_hbm.at[pl.ds(cc * _CH, _CH)], idx_v)
            pltpu.sync_copy(vals_v, acc_sh.at[plsc.Indices(idx_v)], add=True)

        return carry

    lax.fori_loop(0, (_SCH + _NW - 1) // _NW, body, 0)
    plsc.subcore_barrier()
    pltpu.sync_copy(acc_sh.at[pl.ds(sid * stripe, stripe)], buf_v)
    pltpu.sync_copy(buf_v, out_hbm.at[pl.ds(cid * _N + sid * stripe, stripe)])


def _full(shape):
    return pl.BlockSpec(shape, lambda i: tuple(0 for _ in shape))


def kernel(nodes, edges, receivers, senders, bi_edges_indx,
           ne_W1, ne_b1, ne_W2, ne_b2,
           ee_W1, ee_b1, ee_W2, ee_b2,
           me_W1, me_b1, me_W2, me_b2,
           mn_W1, mn_b1, mn_W2, mn_b2,
           dec_W1, dec_b1, dec_W2, dec_b2):
    f32 = jnp.float32
    i32 = jnp.int32
    src = senders[_N:_N + _P]
    dst = receivers[_N:_N + _P]
    sd = jnp.concatenate([src, dst]).astype(i32)
    zeros = jnp.zeros((_N, _H), f32)

    meA = me_W1[:_H]
    meB = me_W1[_H:2 * _H]
    meC = me_W1[2 * _H:]
    mnA = mn_W1[:_H]
    mnG = mn_W1[_H:]
    me_b1r = me_b1.reshape(1, _H)
    me_b2r = me_b2.reshape(1, _H)
    mn_b1r = mn_b1.reshape(1, _H)
    mn_b2r = mn_b2.reshape(1, _H)
    dc_b1r = dec_b1.reshape(1, _H)
    dc_w2b = jnp.tile(dec_W2, (1, _H))
    dc_b2r = jnp.tile(dec_b2.reshape(1, 1), (1, _H))
    ne_w1t = ne_W1.T
    ne_b1c = ne_b1.reshape(_H, 1)
    ne_w2t = ne_W2.T
    ne_b2c = ne_b2.reshape(_H, 1)
    ee_w1t = ee_W1.T
    ee_b1c = ee_b1.reshape(_H, 1)
    ee_w2t = ee_W2.T
    ee_b2c = ee_b2.reshape(_H, 1)

    x1 = edges[_N:_N + _P].reshape(_PSTEP, 1, _BP)
    x2 = edges[_N + _P:].reshape(_PSTEP, 1, _BP)
    x0 = edges[:_N].reshape(_NSTEP, 1, _BN)
    xn = nodes.reshape(_NSTEP, 1, _BN)

    w16 = _full((_H, _H))
    b16 = _full((1, _H))
    wenc = [_full((_H, 1)), _full((_H, 1)), _full((_H, _H)), _full((_H, 1))]

    def rowspec(block, width):
        return pl.BlockSpec((block, width), lambda i: (i, 0))

    enc_pair = pl.pallas_call(
        _enc_body,
        grid=(_PSTEP,),
        in_specs=[pl.BlockSpec((1, 1, _BP), lambda i: (i, 0, 0))] + wenc,
        out_specs=rowspec(_BP, _H),
        out_shape=jax.ShapeDtypeStruct((_P, _H), f32),
    )
    e1 = enc_pair(x1, ee_w1t, ee_b1c, ee_w2t, ee_b2c)
    e2 = enc_pair(x2, ee_w1t, ee_b1c, ee_w2t, ee_b2c)
    e0 = pl.pallas_call(
        _enc_body,
        grid=(_NSTEP,),
        in_specs=[pl.BlockSpec((1, 1, _BN), lambda i: (i, 0, 0))] + wenc,
        out_specs=rowspec(_BN, _H),
        out_shape=jax.ShapeDtypeStruct((_N, _H), f32),
    )(x0, ee_w1t, ee_b1c, ee_w2t, ee_b2c)

    nh, tbl = pl.pallas_call(
        _enc_node_body,
        grid=(_NSTEP,),
        in_specs=[pl.BlockSpec((1, 1, _BN), lambda i: (i, 0, 0))] + wenc
        + [w16, w16],
        out_specs=[rowspec(_BN, _H), rowspec(_BN, 2 * _H)],
        out_shape=[jax.ShapeDtypeStruct((_N, _H), f32),
                   jax.ShapeDtypeStruct((_N, 2 * _H), f32)],
    )(xn, ne_w1t, ne_b1c, ne_w2t, ne_b2c, meB, meC)

    pair_call = pl.pallas_call(
        _pair_body,
        grid=(_PSTEP,),
        in_specs=[rowspec(_BP, _H), rowspec(_BP, _H),
                  pl.BlockSpec((_BP, 2 * _H), lambda i: (i, 0)),
                  pl.BlockSpec((_BP, 2 * _H), lambda i: (i + _PSTEP, 0)),
                  w16, b16, w16, b16],
        out_specs=[rowspec(_BP, _H), rowspec(_BP, _H)],
        out_shape=[jax.ShapeDtypeStruct((_P, _H), f32),
                   jax.ShapeDtypeStruct((_P, _H), f32)],
    )
    node_call = pl.pallas_call(
        _node_body,
        grid=(_NSTEP,),
        in_specs=[rowspec(_BN, _H), rowspec(_BN, _H), rowspec(_BN, 2 * _H),
                  pl.BlockSpec((_BN, _H), lambda i: (i, 0)),
                  pl.BlockSpec((_BN, _H), lambda i: (i + _NSTEP, 0)),
                  w16, b16, w16, b16,
                  w16, w16, b16, w16, b16, w16, w16],
        out_specs=[rowspec(_BN, _H), rowspec(_BN, 2 * _H), rowspec(_BN, _H)],
        out_shape=[jax.ShapeDtypeStruct((_N, _H), f32),
                   jax.ShapeDtypeStruct((_N, 2 * _H), f32),
                   jax.ShapeDtypeStruct((_N, _H), f32)],
    )

    for _ in range(3):
        g = _sc_gather(tbl, sd)
        e1n, e2n = pair_call(e1, e2, g, g, meA, me_b1r, me_W2, me_b2r)
        parts = _sc_scatter(e1n, dst, e2n, src, zeros)
        nh, tbl, e0 = node_call(nh, e0, tbl, parts, parts,
                                meA, me_b1r, me_W2, me_b2r,
                                mnA, mnG, mn_b1r, mn_W2, mn_b2r, meB, meC)
        e1, e2 = e1n, e2n

    dv = pl.pallas_call(
        _dec_body,
        grid=(_PSTEP,),
        in_specs=[rowspec(_BP, _H), rowspec(_BP, _H),
                  w16, b16, w16, b16],
        out_specs=rowspec(_BP, _H),
        out_shape=jax.ShapeDtypeStruct((_P, _H), f32),
    )(e1, e2, dec_W1, dc_b1r, dc_w2b, dc_b2r)[:, 0]

    zi = jnp.zeros((), i32)
    t1 = dst >= src
    t2 = src >= dst
    data = jnp.concatenate([edges[:_N, 0],
                            jnp.where(t1, dv, 0.0),
                            jnp.where(t2, dv, 0.0)])
    rr = jnp.concatenate([receivers[:_N],
                          jnp.where(t1, dst, zi),
                          jnp.where(t2, src, zi)])
    ss = jnp.concatenate([senders[:_N],
                          jnp.where(t1, src, zi),
                          jnp.where(t2, dst, zi)])
    return data, jnp.stack([rr, ss], axis=1)
